# baseline (speedup 1.0000x reference)
import functools

import jax
import jax.numpy as jnp
from jax import lax
from jax.experimental import pallas as pl
from jax.experimental.pallas import tpu as pltpu
from jax.experimental.pallas import tpu_sc as plsc

_NC = 2
_NS = 16
_NW = _NC * _NS


def _tdot_body(wm_ref, wh_ref, tbl_ref, o_ref):
    v_row = lax.dot_general(
        wh_ref[...], wm_ref[...],
        dimension_numbers=(((0,), (1,)), ((), ())),
        preferred_element_type=jnp.float32,
    )
    o_ref[...] = lax.dot_general(
        v_row, tbl_ref[...],
        dimension_numbers=(((1,), (1,)), ((), ())),
        preferred_element_type=jnp.float32,
    )[None]


def _tdot(w_mix, w_head, table):
    k_rows, d = table.shape
    blk = 25000 if k_rows % 25000 == 0 else 16384
    nblk = -(-k_rows // blk)
    out = pl.pallas_call(
        _tdot_body,
        grid=(nblk,),
        in_specs=[
            pl.BlockSpec((d, d), lambda i: (0, 0)),
            pl.BlockSpec((d, 1), lambda i: (0, 0)),
            pl.BlockSpec((blk, d), lambda i: (i, 0)),
        ],
        out_specs=pl.BlockSpec((1, 1, blk), lambda i: (i, 0, 0)),
        out_shape=jax.ShapeDtypeStruct((nblk, 1, blk), jnp.float32),
    )(w_mix, w_head, table)
    return out.reshape(nblk * blk)


def _sc_gather(tdot, ids):
    b = ids.shape[0]
    bpw = b // _NW
    mesh = plsc.VectorSubcoreMesh(core_axis_name="c", subcore_axis_name="s")

    @functools.partial(
        pl.kernel,
        out_type=jax.ShapeDtypeStruct((b,), jnp.float32),
        mesh=mesh,
        scratch_types=[
            pltpu.VMEM((bpw,), jnp.int32),
            pltpu.VMEM((bpw,), jnp.float32),
            pltpu.SemaphoreType.DMA,
        ],
    )
    def gather_kernel(tdot_hbm, ids_hbm, out_hbm, idx_v, s_v, sem):
        wid = lax.axis_index("s") * _NC + lax.axis_index("c")
        base = wid * bpw
        pltpu.sync_copy(ids_hbm.at[pl.ds(base, bpw)], idx_v)
        copies = [
            pltpu.async_copy(
                tdot_hbm.at[idx_v.at[pl.ds(j * 128, 128)]],
                s_v.at[pl.ds(j * 128, 128)],
                sem,
            )
            for j in range(bpw // 128)
        ]
        for cp in copies:
            cp.wait()
        pltpu.sync_copy(s_v, out_hbm.at[pl.ds(base, bpw)])

    return gather_kernel(tdot, ids)


def _head_body(h_ref, wh_ref, b_ref, o_ref):
    o_ref[...] = (
        lax.dot_general(
            wh_ref[...], h_ref[...],
            dimension_numbers=(((0,), (1,)), ((), ())),
            preferred_element_type=jnp.float32,
        )
        + b_ref[0, 0]
    )[None]


def _head(h, w_head, b_head2):
    b, d = h.shape
    blk = 2048
    nblk = b // blk
    return pl.pallas_call(
        _head_body,
        grid=(nblk,),
        in_specs=[
            pl.BlockSpec((blk, d), lambda i: (i, 0)),
            pl.BlockSpec((d, 1), lambda i: (0, 0)),
            pl.BlockSpec((1, 1), lambda i: (0, 0)),
        ],
        out_specs=pl.BlockSpec((1, 1, blk), lambda i: (i, 0, 0)),
        out_shape=jax.ShapeDtypeStruct((nblk, 1, blk), jnp.float32),
    )(h, w_head, b_head2)


def kernel(h, topic_ids, topic_table, W_mix, W_head, b_head):
    b = h.shape[0]
    ids = topic_ids.astype(jnp.int32)
    tdot = _tdot(W_mix, W_head, topic_table)
    s = _sc_gather(tdot, ids)
    hb = _head(h, W_head, b_head.reshape(1, 1))
    nblk, _, blk = hb.shape
    return (hb + s.reshape(nblk, 1, blk)).reshape(b, 1)


# speedup vs baseline: 1.0557x; 1.0557x over previous
import functools

import jax
import jax.numpy as jnp
from jax import lax
from jax.experimental import pallas as pl
from jax.experimental.pallas import tpu as pltpu
from jax.experimental.pallas import tpu_sc as plsc

_NC = 2
_NS = 16
_NW = _NC * _NS


def _tdot_head_body(wm_ref, wh_ref, tbl_ref, h_ref, b_ref, o_ref, hb_ref):
    v_row = lax.dot_general(
        wh_ref[...], wm_ref[...],
        dimension_numbers=(((0,), (1,)), ((), ())),
        preferred_element_type=jnp.float32,
    )
    o_ref[...] = lax.dot_general(
        v_row, tbl_ref[...],
        dimension_numbers=(((1,), (1,)), ((), ())),
        preferred_element_type=jnp.float32,
    )[None]
    hb_ref[...] = (
        lax.dot_general(
            wh_ref[...], h_ref[...],
            dimension_numbers=(((0,), (1,)), ((), ())),
            preferred_element_type=jnp.float32,
        )
        + b_ref[0, 0]
    )[None]


def _tdot_head(w_mix, w_head, table, h, b_head2):
    k_rows, d = table.shape
    b = h.shape[0]
    nblk = 8
    blk = -(-k_rows // (8 * nblk)) * 8
    hblk = b // nblk
    out, hb = pl.pallas_call(
        _tdot_head_body,
        grid=(nblk,),
        in_specs=[
            pl.BlockSpec((d, d), lambda i: (0, 0)),
            pl.BlockSpec((d, 1), lambda i: (0, 0)),
            pl.BlockSpec((blk, d), lambda i: (i, 0)),
            pl.BlockSpec((hblk, d), lambda i: (i, 0)),
            pl.BlockSpec((1, 1), lambda i: (0, 0)),
        ],
        out_specs=(
            pl.BlockSpec((1, 1, blk), lambda i: (i, 0, 0)),
            pl.BlockSpec((1, 1, hblk), lambda i: (i, 0, 0)),
        ),
        out_shape=(
            jax.ShapeDtypeStruct((nblk, 1, blk), jnp.float32),
            jax.ShapeDtypeStruct((nblk, 1, hblk), jnp.float32),
        ),
    )(w_mix, w_head, table, h, b_head2)
    return out.reshape(nblk * blk), hb


def _sc_gather(tdot, ids):
    b = ids.shape[0]
    bpw = b // _NW
    mesh = plsc.VectorSubcoreMesh(core_axis_name="c", subcore_axis_name="s")

    @functools.partial(
        pl.kernel,
        out_type=jax.ShapeDtypeStruct((b,), jnp.float32),
        mesh=mesh,
        scratch_types=[
            pltpu.VMEM((bpw,), jnp.int32),
            pltpu.VMEM((bpw,), jnp.float32),
            pltpu.SemaphoreType.DMA,
        ],
    )
    def gather_kernel(tdot_hbm, ids_hbm, out_hbm, idx_v, s_v, sem):
        wid = lax.axis_index("s") * _NC + lax.axis_index("c")
        base = wid * bpw
        pltpu.sync_copy(ids_hbm.at[pl.ds(base, bpw)], idx_v)
        copies = [
            pltpu.async_copy(
                tdot_hbm.at[idx_v.at[pl.ds(j * 128, 128)]],
                s_v.at[pl.ds(j * 128, 128)],
                sem,
            )
            for j in range(bpw // 128)
        ]
        for cp in copies:
            cp.wait()
        pltpu.sync_copy(s_v, out_hbm.at[pl.ds(base, bpw)])

    return gather_kernel(tdot, ids)


def _head_body(h_ref, wh_ref, b_ref, o_ref):
    o_ref[...] = (
        lax.dot_general(
            wh_ref[...], h_ref[...],
            dimension_numbers=(((0,), (1,)), ((), ())),
            preferred_element_type=jnp.float32,
        )
        + b_ref[0, 0]
    )[None]


def _head(h, w_head, b_head2):
    b, d = h.shape
    blk = 2048
    nblk = b // blk
    return pl.pallas_call(
        _head_body,
        grid=(nblk,),
        in_specs=[
            pl.BlockSpec((blk, d), lambda i: (i, 0)),
            pl.BlockSpec((d, 1), lambda i: (0, 0)),
            pl.BlockSpec((1, 1), lambda i: (0, 0)),
        ],
        out_specs=pl.BlockSpec((1, 1, blk), lambda i: (i, 0, 0)),
        out_shape=jax.ShapeDtypeStruct((nblk, 1, blk), jnp.float32),
    )(h, w_head, b_head2)


def kernel(h, topic_ids, topic_table, W_mix, W_head, b_head):
    b = h.shape[0]
    ids = topic_ids.astype(jnp.int32)
    tdot, hb = _tdot_head(W_mix, W_head, topic_table, h, b_head.reshape(1, 1))
    s = _sc_gather(tdot, ids)
    nblk, _, hblk = hb.shape
    return (hb + s.reshape(nblk, 1, hblk)).reshape(b, 1)


# speedup vs baseline: 1.0708x; 1.0142x over previous
import functools

import jax
import jax.numpy as jnp
from jax import lax
from jax.experimental import pallas as pl
from jax.experimental.pallas import tpu as pltpu
from jax.experimental.pallas import tpu_sc as plsc

_NC = 2
_NS = 16
_NW = _NC * _NS


def _tdot_body(wm_ref, wh_ref, tbl_ref, o_ref):
    v_row = lax.dot_general(
        wh_ref[...], wm_ref[...],
        dimension_numbers=(((0,), (1,)), ((), ())),
        preferred_element_type=jnp.float32,
    )
    o_ref[...] = lax.dot_general(
        v_row, tbl_ref[...],
        dimension_numbers=(((1,), (1,)), ((), ())),
        preferred_element_type=jnp.float32,
    )[None]


def _tdot(w_mix, w_head, table):
    k_rows, d = table.shape
    blk = 16384
    nblk = -(-k_rows // blk)
    out = pl.pallas_call(
        _tdot_body,
        grid=(nblk,),
        in_specs=[
            pl.BlockSpec((d, d), lambda i: (0, 0)),
            pl.BlockSpec((d, 1), lambda i: (0, 0)),
            pl.BlockSpec((blk, d), lambda i: (i, 0)),
        ],
        out_specs=pl.BlockSpec((1, 1, blk), lambda i: (i, 0, 0)),
        out_shape=jax.ShapeDtypeStruct((nblk, 1, blk), jnp.float32),
    )(w_mix, w_head, table)
    return out.reshape(nblk * blk)


def _sc_gather(tdot, ids):
    b = ids.shape[0]
    bpw = b // _NW
    mesh = plsc.VectorSubcoreMesh(core_axis_name="c", subcore_axis_name="s")

    @functools.partial(
        pl.kernel,
        out_type=jax.ShapeDtypeStruct((b,), jnp.float32),
        mesh=mesh,
        scratch_types=[
            pltpu.VMEM((bpw,), jnp.int32),
            pltpu.VMEM((bpw,), jnp.float32),
            pltpu.SemaphoreType.DMA,
        ],
    )
    def gather_kernel(tdot_hbm, ids_hbm, out_hbm, idx_v, s_v, sem):
        wid = lax.axis_index("s") * _NC + lax.axis_index("c")
        base = wid * bpw
        pltpu.sync_copy(ids_hbm.at[pl.ds(base, bpw)], idx_v)
        pltpu.async_copy(tdot_hbm.at[idx_v], s_v, sem).wait()
        pltpu.sync_copy(s_v, out_hbm.at[pl.ds(base, bpw)])

    return gather_kernel(tdot, ids)


def _head_body(h_ref, wh_ref, b_ref, o_ref):
    o_ref[...] = (
        lax.dot_general(
            wh_ref[...], h_ref[...],
            dimension_numbers=(((0,), (1,)), ((), ())),
            preferred_element_type=jnp.float32,
        )
        + b_ref[0, 0]
    )[None]


def _head(h, w_head, b_head2):
    b, d = h.shape
    blk = 2048
    nblk = b // blk
    return pl.pallas_call(
        _head_body,
        grid=(nblk,),
        in_specs=[
            pl.BlockSpec((blk, d), lambda i: (i, 0)),
            pl.BlockSpec((d, 1), lambda i: (0, 0)),
            pl.BlockSpec((1, 1), lambda i: (0, 0)),
        ],
        out_specs=pl.BlockSpec((1, 1, blk), lambda i: (i, 0, 0)),
        out_shape=jax.ShapeDtypeStruct((nblk, 1, blk), jnp.float32),
    )(h, w_head, b_head2)


def kernel(h, topic_ids, topic_table, W_mix, W_head, b_head):
    b = h.shape[0]
    ids = topic_ids.astype(jnp.int32)
    tdot = _tdot(W_mix, W_head, topic_table)
    s = _sc_gather(tdot, ids)
    hb = _head(h, W_head, b_head.reshape(1, 1))
    nblk, _, blk = hb.shape
    return (hb + s.reshape(nblk, 1, blk)).reshape(b, 1)
